# Initial kernel scaffold; baseline (speedup 1.0000x reference)
import jax
import jax.numpy as jnp
from jax.experimental import pallas as pl


def kernel(fine_features, coarse_features, global_features, W_f2c, b_f2c, g_f2c, be_f2c, W_c2f, b_c2f, g_c2f, be_c2f, W_gate, b_gate, W_gi, b_gi, g_gi, be_gi, atom_to_coarse):
    raise NotImplementedError("write your pallas kernel here")


# speedup vs baseline: 3.7404x; 3.7404x over previous
import functools

import jax
import jax.numpy as jnp
from jax.experimental import pallas as pl


_BB = 8


def _ln(x, g, b, eps=1e-5):
    mu = jnp.mean(x, axis=-1, keepdims=True)
    xc = x - mu
    var = jnp.mean(xc * xc, axis=-1, keepdims=True)
    return xc * jax.lax.rsqrt(var + eps) * g + b


def _encoder_kernel(idx_ref, fine_ref, coarse_ref, glob_ref,
                    W_f2c_ref, W_c2f_ref, W_gate_ref, W_gi_ref,
                    b_f2c_ref, g_f2c_ref, be_f2c_ref,
                    b_c2f_ref, g_c2f_ref, be_c2f_ref,
                    b_gate_ref, b_gi_ref, g_gi_ref, be_gi_ref,
                    fine_out_ref, coarse_out_ref):
    BB, A, D = fine_ref.shape
    C = coarse_ref.shape[1]
    f32 = jnp.float32

    Wf2c = W_f2c_ref[...]
    Wc2f = W_c2f_ref[...]
    Wg1 = W_gate_ref[:D, :]
    Wg2 = W_gate_ref[D:, :]
    Wgi1 = W_gi_ref[:D, :]
    Wgi2 = W_gi_ref[D:, :]
    b_f2c = b_f2c_ref[...]
    g_f2c = g_f2c_ref[...]
    be_f2c = be_f2c_ref[...]
    b_c2f = b_c2f_ref[...]
    g_c2f = g_c2f_ref[...]
    be_c2f = be_c2f_ref[...]
    b_gate = b_gate_ref[...]
    b_gi = b_gi_ref[...]
    g_gi = g_gi_ref[...]
    be_gi = be_gi_ref[...]

    def mm(x, w):
        return jax.lax.dot_general(x, w, (((1,), (0,)), ((), ())),
                                   preferred_element_type=f32)

    for b in range(BB):
        fine = fine_ref[b]
        coarse = coarse_ref[b]
        glob = glob_ref[b]
        idx2 = idx_ref[b, :][None, :]

        onehotT = (idx2 == jax.lax.broadcasted_iota(jnp.int32, (C, A), 0)
                   ).astype(f32)
        seg = mm(onehotT, fine)
        counts = jnp.sum(onehotT, axis=1, keepdims=True)
        cff = seg / jnp.maximum(counts, 1.0)
        cff = jax.nn.relu(_ln(mm(cff, Wf2c) + b_f2c, g_f2c, be_f2c))

        ffc = jax.lax.dot_general(onehotT, coarse, (((0,), (0,)), ((), ())),
                                  preferred_element_type=f32)
        ffc = jax.nn.relu(_ln(mm(ffc, Wc2f) + b_c2f, g_c2f, be_c2f))

        fg = jax.nn.sigmoid(mm(fine, Wg1) + mm(ffc, Wg2) + b_gate)
        fine_upd = fg * fine + (1.0 - fg) * ffc

        cg = jax.nn.sigmoid(mm(coarse, Wg1) + mm(cff, Wg2) + b_gate)
        coarse_upd = cg * coarse + (1.0 - cg) * cff

        fwg = jax.nn.relu(_ln(mm(fine_upd, Wgi1) + mm(glob, Wgi2) + b_gi,
                              g_gi, be_gi))
        fine_out_ref[b] = fine_upd + 0.1 * fwg

        gmean = jnp.mean(glob, axis=0, keepdims=True)
        gterm = mm(gmean, Wgi2)
        cwg = jax.nn.relu(_ln(mm(coarse_upd, Wgi1) + gterm + b_gi,
                              g_gi, be_gi))
        coarse_out_ref[b] = coarse_upd + 0.1 * cwg


def _specs(B, A, C, D, BB):
    grid = (B // BB,)

    def blk(i):
        return (i, 0, 0)

    def rep2(i):
        return (0, 0)

    in_specs = [
        pl.BlockSpec((BB, A), lambda i: (i, 0)),
        pl.BlockSpec((BB, A, D), blk),
        pl.BlockSpec((BB, C, D), blk),
        pl.BlockSpec((BB, A, D), blk),
        pl.BlockSpec((D, D), rep2),
        pl.BlockSpec((D, D), rep2),
        pl.BlockSpec((2 * D, D), rep2),
        pl.BlockSpec((2 * D, D), rep2),
    ] + [pl.BlockSpec((1, D), rep2)] * 10
    out_specs = [
        pl.BlockSpec((BB, A, D), blk),
        pl.BlockSpec((BB, C, D), blk),
    ]
    return grid, in_specs, out_specs


def kernel(fine_features, coarse_features, global_features,
           W_f2c, b_f2c, g_f2c, be_f2c,
           W_c2f, b_c2f, g_c2f, be_c2f,
           W_gate, b_gate, W_gi, b_gi, g_gi, be_gi, atom_to_coarse):
    B, A, D = fine_features.shape
    C = coarse_features.shape[1]
    BB = _BB
    grid, in_specs, out_specs = _specs(B, A, C, D, BB)
    vecs = [b_f2c, g_f2c, be_f2c, b_c2f, g_c2f, be_c2f,
            b_gate, b_gi, g_gi, be_gi]
    vecs = [v.reshape(1, D) for v in vecs]
    out_shape = [
        jax.ShapeDtypeStruct((B, A, D), fine_features.dtype),
        jax.ShapeDtypeStruct((B, C, D), coarse_features.dtype),
    ]
    fine_out, coarse_out = pl.pallas_call(
        _encoder_kernel,
        grid=grid,
        in_specs=in_specs,
        out_specs=out_specs,
        out_shape=out_shape,
    )(atom_to_coarse.astype(jnp.int32), fine_features, coarse_features,
      global_features, W_f2c, W_c2f, W_gate, W_gi, *vecs)
    return (fine_out, coarse_out)


# speedup vs baseline: 4.7873x; 1.2799x over previous
import jax
import jax.numpy as jnp
from jax.experimental import pallas as pl


_BB = 8


def _ln(x, g, b, eps=1e-5):
    mu = jnp.mean(x, axis=-1, keepdims=True)
    xc = x - mu
    var = jnp.mean(xc * xc, axis=-1, keepdims=True)
    return xc * jax.lax.rsqrt(var + eps) * g + b


def _make_body(BB, A, C, D):
    BBA = BB * A
    BBC = BB * C
    f32 = jnp.float32

    def mm(x, w):
        return jax.lax.dot_general(x, w, (((1,), (0,)), ((), ())),
                                   preferred_element_type=f32)

    def body(gidx_ref, fine_ref, coarse_ref, glob_ref,
             W_f2c_ref, W_c2f_ref, W_gate_ref, W_gi_ref,
             b_f2c_ref, g_f2c_ref, be_f2c_ref,
             b_c2f_ref, g_c2f_ref, be_c2f_ref,
             b_gate_ref, b_gi_ref, g_gi_ref, be_gi_ref,
             fine_out_ref, coarse_out_ref):
        Wf2c = W_f2c_ref[...]
        Wc2f = W_c2f_ref[...]
        Wg1 = W_gate_ref[:D, :]
        Wg2 = W_gate_ref[D:, :]
        Wgi1 = W_gi_ref[:D, :]
        Wgi2 = W_gi_ref[D:, :]

        fine = fine_ref[...]
        coarse = coarse_ref[...]
        glob = glob_ref[...]
        gidx = gidx_ref[0]

        seg_iota = jax.lax.broadcasted_iota(jnp.int32, (BBC, BBA), 0)
        onehotT = (gidx == seg_iota).astype(f32)

        seg = mm(onehotT, fine)
        counts = jnp.sum(onehotT, axis=1, keepdims=True)
        cff = seg / jnp.maximum(counts, 1.0)
        cff = jax.nn.relu(_ln(mm(cff, Wf2c) + b_f2c_ref[...],
                              g_f2c_ref[...], be_f2c_ref[...]))

        ffc = jax.lax.dot_general(onehotT, coarse,
                                  (((0,), (0,)), ((), ())),
                                  preferred_element_type=f32)
        ffc = jax.nn.relu(_ln(mm(ffc, Wc2f) + b_c2f_ref[...],
                              g_c2f_ref[...], be_c2f_ref[...]))

        b_gate = b_gate_ref[...]
        fg = jax.nn.sigmoid(mm(fine, Wg1) + mm(ffc, Wg2) + b_gate)
        fine_upd = fg * fine + (1.0 - fg) * ffc

        cg = jax.nn.sigmoid(mm(coarse, Wg1) + mm(cff, Wg2) + b_gate)
        coarse_upd = cg * coarse + (1.0 - cg) * cff

        b_gi = b_gi_ref[...]
        g_gi = g_gi_ref[...]
        be_gi = be_gi_ref[...]
        fwg = jax.nn.relu(_ln(mm(fine_upd, Wgi1) + mm(glob, Wgi2) + b_gi,
                              g_gi, be_gi))
        fine_out_ref[...] = fine_upd + 0.1 * fwg

        meanmat = (jax.lax.broadcasted_iota(jnp.int32, (BB, BBA), 1) // A ==
                   jax.lax.broadcasted_iota(jnp.int32, (BB, BBA), 0)
                   ).astype(f32) * (1.0 / A)
        gterm = mm(mm(meanmat, glob), Wgi2)
        expmat = (jax.lax.broadcasted_iota(jnp.int32, (BBC, BB), 0) // C ==
                  jax.lax.broadcasted_iota(jnp.int32, (BBC, BB), 1)
                  ).astype(f32)
        gterm_x = mm(expmat, gterm)

        cwg = jax.nn.relu(_ln(mm(coarse_upd, Wgi1) + gterm_x + b_gi,
                              g_gi, be_gi))
        coarse_out_ref[...] = coarse_upd + 0.1 * cwg

    return body


def _specs(B, A, C, D, BB):
    grid = (B // BB,)

    def blk(i):
        return (i, 0)

    def rep2(i):
        return (0, 0)

    in_specs = [
        pl.BlockSpec((1, 1, BB * A), lambda i: (i, 0, 0)),
        pl.BlockSpec((BB * A, D), blk),
        pl.BlockSpec((BB * C, D), blk),
        pl.BlockSpec((BB * A, D), blk),
        pl.BlockSpec((D, D), rep2),
        pl.BlockSpec((D, D), rep2),
        pl.BlockSpec((2 * D, D), rep2),
        pl.BlockSpec((2 * D, D), rep2),
    ] + [pl.BlockSpec((1, D), rep2)] * 10
    out_specs = [
        pl.BlockSpec((BB * A, D), blk),
        pl.BlockSpec((BB * C, D), blk),
    ]
    return grid, in_specs, out_specs


def kernel(fine_features, coarse_features, global_features,
           W_f2c, b_f2c, g_f2c, be_f2c,
           W_c2f, b_c2f, g_c2f, be_c2f,
           W_gate, b_gate, W_gi, b_gi, g_gi, be_gi, atom_to_coarse):
    B, A, D = fine_features.shape
    C = coarse_features.shape[1]
    BB = _BB
    grid, in_specs, out_specs = _specs(B, A, C, D, BB)
    vecs = [b_f2c, g_f2c, be_f2c, b_c2f, g_c2f, be_c2f,
            b_gate, b_gi, g_gi, be_gi]
    vecs = [v.reshape(1, D) for v in vecs]
    offs = (jnp.arange(B, dtype=jnp.int32) % BB) * C
    gidx = (atom_to_coarse.astype(jnp.int32) + offs[:, None]
            ).reshape(B // BB, 1, BB * A)
    out_shape = [
        jax.ShapeDtypeStruct((B * A, D), fine_features.dtype),
        jax.ShapeDtypeStruct((B * C, D), coarse_features.dtype),
    ]
    fine_out, coarse_out = pl.pallas_call(
        _make_body(BB, A, C, D),
        grid=grid,
        in_specs=in_specs,
        out_specs=out_specs,
        out_shape=out_shape,
    )(gidx, fine_features.reshape(B * A, D),
      coarse_features.reshape(B * C, D),
      global_features.reshape(B * A, D),
      W_f2c, W_c2f, W_gate, W_gi, *vecs)
    return (fine_out.reshape(B, A, D), coarse_out.reshape(B, C, D))


# speedup vs baseline: 6.4477x; 1.3468x over previous
import jax
import jax.numpy as jnp
from jax.experimental import pallas as pl


_BB = 8


def _ln(x, g, b, eps=1e-5):
    mu = jnp.mean(x, axis=-1, keepdims=True)
    xc = x - mu
    var = jnp.mean(xc * xc, axis=-1, keepdims=True)
    return xc * jax.lax.rsqrt(var + eps) * g + b


def _make_body(BB, A, C, D):
    BBA = BB * A
    BBC = BB * C
    f32 = jnp.float32

    def mm(x, w):
        return jax.lax.dot_general(x, w, (((1,), (0,)), ((), ())),
                                   preferred_element_type=f32)

    def body(idx_ref, fine_ref, coarse_ref, glob_ref,
             W_f2c_ref, W_c2f_ref, W_gate_ref, W_gi_ref,
             b_f2c_ref, g_f2c_ref, be_f2c_ref,
             b_c2f_ref, g_c2f_ref, be_c2f_ref,
             b_gate_ref, b_gi_ref, g_gi_ref, be_gi_ref,
             fine_out_ref, coarse_out_ref):
        Wf2c = W_f2c_ref[...]
        Wc2f = W_c2f_ref[...]
        Wg1 = W_gate_ref[:D, :]
        Wg2 = W_gate_ref[D:, :]
        Wgi1 = W_gi_ref[:D, :]
        Wgi2 = W_gi_ref[D:, :]

        fine = jnp.concatenate([fine_ref[b] for b in range(BB)], axis=0)
        coarse = jnp.concatenate([coarse_ref[b] for b in range(BB)], axis=0)
        glob = jnp.concatenate([glob_ref[b] for b in range(BB)], axis=0)
        idx_flat = jnp.concatenate(
            [idx_ref[b:b + 1, :] for b in range(BB)], axis=1)
        col_mol = jax.lax.broadcasted_iota(jnp.int32, (1, BBA), 1) // A
        gidx = idx_flat + col_mol * C

        seg_iota = jax.lax.broadcasted_iota(jnp.int32, (BBC, BBA), 0)
        onehotT = (gidx == seg_iota).astype(f32)

        seg = mm(onehotT, fine)
        counts = jnp.sum(onehotT, axis=1, keepdims=True)
        cff = seg / jnp.maximum(counts, 1.0)
        cff = jax.nn.relu(_ln(mm(cff, Wf2c) + b_f2c_ref[...],
                              g_f2c_ref[...], be_f2c_ref[...]))

        ffc = jax.lax.dot_general(onehotT, coarse,
                                  (((0,), (0,)), ((), ())),
                                  preferred_element_type=f32)
        ffc = jax.nn.relu(_ln(mm(ffc, Wc2f) + b_c2f_ref[...],
                              g_c2f_ref[...], be_c2f_ref[...]))

        b_gate = b_gate_ref[...]
        fg = jax.nn.sigmoid(mm(fine, Wg1) + mm(ffc, Wg2) + b_gate)
        fine_upd = fg * fine + (1.0 - fg) * ffc

        cg = jax.nn.sigmoid(mm(coarse, Wg1) + mm(cff, Wg2) + b_gate)
        coarse_upd = cg * coarse + (1.0 - cg) * cff

        b_gi = b_gi_ref[...]
        g_gi = g_gi_ref[...]
        be_gi = be_gi_ref[...]
        fwg = jax.nn.relu(_ln(mm(fine_upd, Wgi1) + mm(glob, Wgi2) + b_gi,
                              g_gi, be_gi))
        fine_out = fine_upd + 0.1 * fwg

        meanmat = (jax.lax.broadcasted_iota(jnp.int32, (BB, BBA), 1) // A ==
                   jax.lax.broadcasted_iota(jnp.int32, (BB, BBA), 0)
                   ).astype(f32) * (1.0 / A)
        gterm = mm(mm(meanmat, glob), Wgi2)
        expmat = (jax.lax.broadcasted_iota(jnp.int32, (BBC, BB), 0) // C ==
                  jax.lax.broadcasted_iota(jnp.int32, (BBC, BB), 1)
                  ).astype(f32)
        gterm_x = mm(expmat, gterm)

        cwg = jax.nn.relu(_ln(mm(coarse_upd, Wgi1) + gterm_x + b_gi,
                              g_gi, be_gi))
        coarse_out = coarse_upd + 0.1 * cwg

        for b in range(BB):
            fine_out_ref[b] = fine_out[b * A:(b + 1) * A, :]
            coarse_out_ref[b] = coarse_out[b * C:(b + 1) * C, :]

    return body


def _specs(B, A, C, D, BB):
    grid = (B // BB,)

    def blk3(i):
        return (i, 0, 0)

    def rep2(i):
        return (0, 0)

    in_specs = [
        pl.BlockSpec((BB, A), lambda i: (i, 0)),
        pl.BlockSpec((BB, A, D), blk3),
        pl.BlockSpec((BB, C, D), blk3),
        pl.BlockSpec((BB, A, D), blk3),
        pl.BlockSpec((D, D), rep2),
        pl.BlockSpec((D, D), rep2),
        pl.BlockSpec((2 * D, D), rep2),
        pl.BlockSpec((2 * D, D), rep2),
    ] + [pl.BlockSpec((1, D), rep2)] * 10
    out_specs = [
        pl.BlockSpec((BB, A, D), blk3),
        pl.BlockSpec((BB, C, D), blk3),
    ]
    return grid, in_specs, out_specs


def kernel(fine_features, coarse_features, global_features,
           W_f2c, b_f2c, g_f2c, be_f2c,
           W_c2f, b_c2f, g_c2f, be_c2f,
           W_gate, b_gate, W_gi, b_gi, g_gi, be_gi, atom_to_coarse):
    B, A, D = fine_features.shape
    C = coarse_features.shape[1]
    BB = _BB
    grid, in_specs, out_specs = _specs(B, A, C, D, BB)
    vecs = [b_f2c, g_f2c, be_f2c, b_c2f, g_c2f, be_c2f,
            b_gate, b_gi, g_gi, be_gi]
    vecs = [v.reshape(1, D) for v in vecs]
    out_shape = [
        jax.ShapeDtypeStruct((B, A, D), fine_features.dtype),
        jax.ShapeDtypeStruct((B, C, D), coarse_features.dtype),
    ]
    fine_out, coarse_out = pl.pallas_call(
        _make_body(BB, A, C, D),
        grid=grid,
        in_specs=in_specs,
        out_specs=out_specs,
        out_shape=out_shape,
    )(atom_to_coarse.astype(jnp.int32), fine_features, coarse_features,
      global_features, W_f2c, W_c2f, W_gate, W_gi, *vecs)
    return (fine_out, coarse_out)
